# baseline (speedup 1.0000x reference)
import functools

import jax
import jax.numpy as jnp
from jax import lax
from jax.experimental import pallas as pl
from jax.experimental.pallas import tpu as pltpu
from jax.experimental.pallas import tpu_sc as plsc

_B = 16384
_D = 64
_NC, _NS = 2, 16
_NW = _NC * _NS
_BPW = _B // _NW


def _body(uids, iids, ut, it, uout, iout, uidx, iidx, usem, isem):
    wid = lax.axis_index("s") * _NC + lax.axis_index("c")
    base = wid * _BPW
    pltpu.sync_copy(uids.at[pl.ds(base, _BPW)], uidx)
    pltpu.sync_copy(iids.at[pl.ds(base, _BPW)], iidx)

    def step(g, carry):
        row = base + g * 16
        uvec = uidx[pl.ds(g * 16, 16)]
        ivec = iidx[pl.ds(g * 16, 16)]
        for j in range(16):
            pltpu.async_copy(ut.at[uvec[j]], uout.at[row + j], usem)
            pltpu.async_copy(it.at[ivec[j]], iout.at[row + j], isem)
        return carry

    lax.fori_loop(0, _BPW // 16, step, 0)
    pltpu.make_async_copy(
        ut.at[pl.ds(0, _BPW)], uout.at[pl.ds(base, _BPW)], usem
    ).wait()
    pltpu.make_async_copy(
        it.at[pl.ds(0, _BPW)], iout.at[pl.ds(base, _BPW)], isem
    ).wait()


_gather = functools.partial(
    pl.kernel,
    mesh=plsc.VectorSubcoreMesh(core_axis_name="c", subcore_axis_name="s"),
    out_type=(
        jax.ShapeDtypeStruct((_B, _D), jnp.float32),
        jax.ShapeDtypeStruct((_B, _D), jnp.float32),
    ),
    scratch_types=[
        pltpu.VMEM((_BPW,), jnp.int32),
        pltpu.VMEM((_BPW,), jnp.int32),
        pltpu.SemaphoreType.DMA,
        pltpu.SemaphoreType.DMA,
    ],
)(_body)


def kernel(user_ids, item_ids, user_table, item_table):
    return _gather(
        user_ids.astype(jnp.int32),
        item_ids.astype(jnp.int32),
        user_table,
        item_table,
    )


# speedup vs baseline: 2.1818x; 2.1818x over previous
import functools

import jax
import jax.numpy as jnp
from jax import lax
from jax.experimental import pallas as pl
from jax.experimental.pallas import tpu as pltpu
from jax.experimental.pallas import tpu_sc as plsc

_B = 16384
_D = 64
_NC, _NS = 2, 16
_NW = _NC * _NS
_BPW = _B // _NW
_CH = 256


def _body(uids, iids, ut, it, uout, iout, uidx, iidx, urows, irows, usem, isem):
    wid = lax.axis_index("s") * _NC + lax.axis_index("c")
    base = wid * _BPW
    pltpu.sync_copy(uids.at[pl.ds(base, _BPW)], uidx)
    pltpu.sync_copy(iids.at[pl.ds(base, _BPW)], iidx)

    def chunk(c, carry):
        cbase = c * _CH

        def step(g, carry2):
            off = cbase + g * 16
            uvec = uidx[pl.ds(off, 16)]
            ivec = iidx[pl.ds(off, 16)]
            row = g * 16
            for j in range(16):
                pltpu.async_copy(ut.at[uvec[j]], urows.at[row + j], usem)
                pltpu.async_copy(it.at[ivec[j]], irows.at[row + j], isem)
            return carry2

        lax.fori_loop(0, _CH // 16, step, 0)
        pltpu.make_async_copy(ut.at[pl.ds(0, _CH)], urows, usem).wait()
        pltpu.sync_copy(urows, uout.at[pl.ds(base + cbase, _CH)])
        pltpu.make_async_copy(it.at[pl.ds(0, _CH)], irows, isem).wait()
        pltpu.sync_copy(irows, iout.at[pl.ds(base + cbase, _CH)])
        return carry

    lax.fori_loop(0, _BPW // _CH, chunk, 0)


_gather = functools.partial(
    pl.kernel,
    mesh=plsc.VectorSubcoreMesh(core_axis_name="c", subcore_axis_name="s"),
    out_type=(
        jax.ShapeDtypeStruct((_B, _D), jnp.float32),
        jax.ShapeDtypeStruct((_B, _D), jnp.float32),
    ),
    scratch_types=[
        pltpu.VMEM((_BPW,), jnp.int32),
        pltpu.VMEM((_BPW,), jnp.int32),
        pltpu.VMEM((_CH, _D), jnp.float32),
        pltpu.VMEM((_CH, _D), jnp.float32),
        pltpu.SemaphoreType.DMA,
        pltpu.SemaphoreType.DMA,
    ],
)(_body)


def kernel(user_ids, item_ids, user_table, item_table):
    return _gather(
        user_ids.astype(jnp.int32),
        item_ids.astype(jnp.int32),
        user_table,
        item_table,
    )
